# baseline (speedup 1.0000x reference)
import functools

import jax
import jax.numpy as jnp
from jax.experimental import pallas as pl
from jax.experimental.pallas import tpu as pltpu
from jax.experimental.pallas import tpu_sc as plsc

_L, _N, _M, _K = 32, 1024, 128, 25
_PLANES = 3
_NC, _NS, _LANES = 2, 16, 16


def _r(x):
    u = jax.lax.bitcast_convert_type(x, jnp.uint32)
    r = (u + jnp.uint32(0x7FFF) + ((u >> 16) & jnp.uint32(1))) \
        & jnp.uint32(0xFFFF0000)
    return jax.lax.bitcast_convert_type(r, jnp.float32)


def _tc_body(ev_ref, w1_ref, b1_ref, w2_ref, b2_ref, w3_ref, b3_ref, e0_ref,
             e1_ref, f_ref):
    l = pl.program_id(0)
    e = ev_ref[pl.ds(l, 1), :]
    h1 = [jnp.maximum(e * w1_ref[p, 0] + b1_ref[p], 0.0) for p in range(3)]
    h1 = [_r(h) for h in h1]
    h2 = []
    for g in range(5):
        t = (h1[0] * w2_ref[g, 0] + h1[1] * w2_ref[g, 1]
             + h1[2] * w2_ref[g, 2] + b2_ref[g])
        h2.append(jnp.maximum(t, 0.0))
    h2 = [_r(h) for h in h2]
    gv = (h2[0] * w3_ref[0, 0] + h2[1] * w3_ref[0, 1] + h2[2] * w3_ref[0, 2]
          + h2[3] * w3_ref[0, 3] + h2[4] * w3_ref[0, 4] + b3_ref[0])
    mu = jnp.mean(gv)
    dev = gv - mu
    var = jnp.mean(dev * dev)
    gn = dev / jnp.sqrt(var + 1e-5)
    gnb = gn.astype(jnp.bfloat16)
    half = _N // 2
    for h, eref in enumerate((e0_ref, e1_ref)):
        emat = eref[0, 0]
        f_ref[pl.ds(l, 1), pl.ds(h * half, half)] = jax.lax.dot_general(
            gnb, emat.astype(jnp.bfloat16), (((1,), (1,)), ((), ())),
            preferred_element_type=jnp.float32)


def _tc_f(ev2, w1, b1, w2, b2, w3, b3, evecs):
    smem = pl.BlockSpec(memory_space=pltpu.SMEM)
    e4 = evecs.reshape(_L, 2, _N // 2, _N)
    eblk = (1, 1, _N // 2, _N)
    return pl.pallas_call(
        _tc_body,
        grid=(_L,),
        in_specs=[
            pl.BlockSpec((_L, _N), lambda l: (0, 0)),
            smem, smem, smem, smem, smem, smem,
            pl.BlockSpec(eblk, lambda l: (l, 0, 0, 0)),
            pl.BlockSpec(eblk, lambda l: (l, 1, 0, 0)),
        ],
        out_specs=pl.BlockSpec((_L, _N), lambda l: (0, 0)),
        out_shape=jax.ShapeDtypeStruct((_L, _N), jnp.float32),
    )(ev2, w1, b1, w2, b2, w3, b3, e4, e4)


def _sc_body(f_hbm, bi_hbm, di_hbm, wpb_hbm, out_hbm,
             f_v, bi_v, di_v, b_v, d_v, wp_v, o_v):
    c = jax.lax.axis_index("c")
    s = jax.lax.axis_index("s")
    wid = s * _NC + c
    pltpu.sync_copy(f_hbm.at[wid], f_v)
    pltpu.sync_copy(bi_hbm.at[wid], bi_v)
    pltpu.sync_copy(di_hbm.at[wid], di_v)
    pltpu.sync_copy(wpb_hbm, wp_v)

    wpvec = wp_v[...]
    lanes = jax.lax.iota(jnp.int32, _LANES)
    neg_inf = jnp.full((_LANES,), -jnp.inf, jnp.float32)
    nchunks = _M // _LANES
    acc = jnp.zeros((_LANES,), jnp.float32)

    for plane in range(_PLANES):
        p_regs = []
        for j in range(nchunks):
            off = plane * _M + j * _LANES
            bi = bi_v[pl.ds(off, _LANES)]
            di = di_v[pl.ds(off, _LANES)]
            bvals = plsc.load_gather(f_v, [bi])
            dvals = plsc.load_gather(f_v, [di])
            b_v[pl.ds(j * _LANES, _LANES)] = bvals
            d_v[pl.ds(j * _LANES, _LANES)] = dvals
            p_regs.append(jnp.abs(dvals - bvals))

        def step(k, carry):
            ps = list(carry[:nchunks])
            prev_b, prev_d, first_b, first_d, cross = carry[nchunks:]
            m = ps[0]
            for j in range(1, nchunks):
                m = jnp.maximum(m, ps[j])
            mb = jnp.broadcast_to(jnp.max(m), (_LANES,))
            selv = jnp.full((_LANES,), 16384, jnp.int32)
            for j in range(nchunks):
                cand = jnp.where(ps[j] == mb, lanes + (j * _LANES), selv)
                selv = jnp.minimum(selv, cand)
            selb = jnp.broadcast_to(jnp.min(selv), (_LANES,))
            bs = plsc.load_gather(b_v, [selb])
            ds_ = plsc.load_gather(d_v, [selb])
            for j in range(nchunks):
                hit = (lanes + (j * _LANES)) == selb
                ps[j] = jnp.where(hit, neg_inf, ps[j])
            cross = cross + bs * prev_d - ds_ * prev_b
            is0 = jnp.broadcast_to(k == 0, (_LANES,))
            first_b = jnp.where(is0, bs, first_b)
            first_d = jnp.where(is0, ds_, first_d)
            return tuple(ps) + (bs, ds_, first_b, first_d, cross)

        z = jnp.zeros((_LANES,), jnp.float32)
        fin = jax.lax.fori_loop(0, _K, step, tuple(p_regs) + (z, z, z, z, z))
        last_b, last_d, first_b, first_d, cross = fin[nchunks:]
        incx = first_b - last_b
        incy = first_d - last_d
        area = 0.5 * (cross - (last_b * first_d - last_d * first_b))
        acc = (acc + incx * wpvec[3 * plane] + incy * wpvec[3 * plane + 1]
               + area * wpvec[3 * plane + 2])

    o_v[...] = acc + wpvec[3 * _PLANES]
    pltpu.sync_copy(o_v, out_hbm.at[wid])


def _sc_call(f, bidx2, didx2, wpb):
    mesh = plsc.VectorSubcoreMesh(core_axis_name="c", subcore_axis_name="s")
    run = functools.partial(
        pl.kernel,
        out_type=jax.ShapeDtypeStruct((_L, _LANES), jnp.float32),
        mesh=mesh,
        compiler_params=pltpu.CompilerParams(needs_layout_passes=False),
        scratch_types=[
            pltpu.VMEM((_N,), jnp.float32),
            pltpu.VMEM((_PLANES * _M,), jnp.int32),
            pltpu.VMEM((_PLANES * _M,), jnp.int32),
            pltpu.VMEM((_M,), jnp.float32),
            pltpu.VMEM((_M,), jnp.float32),
            pltpu.VMEM((_LANES,), jnp.float32),
            pltpu.VMEM((_LANES,), jnp.float32),
        ],
    )(_sc_body)
    return run(f, bidx2, didx2, wpb)


def kernel(eigenvalues, eigenvectors_sq, W1, b1, W2, b2, W3, b3, Wp, bp,
           birth_idx, death_idx):
    ev2 = eigenvalues.reshape(_L, _N)
    w1r = W1
    w2r = _r(W2)
    w3r = _r(W3)
    f = _tc_f(ev2, w1r, b1, w2r, b2, w3r, b3, eigenvectors_sq)
    bidx2 = birth_idx.reshape(_L, _PLANES * _M)
    didx2 = death_idx.reshape(_L, _PLANES * _M)
    wpb = jnp.concatenate([Wp.reshape(-1), bp.reshape(-1),
                           jnp.zeros(_LANES - 3 * _PLANES - 1, jnp.float32)])
    o = _sc_call(f, bidx2, didx2, wpb)
    return o[:, :1]


# speedup vs baseline: 1.0596x; 1.0596x over previous
import functools

import jax
import jax.numpy as jnp
from jax.experimental import pallas as pl
from jax.experimental.pallas import tpu as pltpu
from jax.experimental.pallas import tpu_sc as plsc

_L, _N, _M, _K = 32, 1024, 128, 25
_PLANES = 3
_NC, _NS, _LANES = 2, 16, 16
_M0 = 768
_RH = _N - _M0
_W = 16


def _r(x):
    u = jax.lax.bitcast_convert_type(x, jnp.uint32)
    r = (u + jnp.uint32(0x7FFF) + ((u >> 16) & jnp.uint32(1))) \
        & jnp.uint32(0xFFFF0000)
    return jax.lax.bitcast_convert_type(r, jnp.float32)


def _gn_body(ev_ref, w1_ref, b1_ref, w2_ref, b2_ref, w3_ref, b3_ref, gn_ref):
    e = ev_ref[...]
    h1 = [jnp.maximum(e * w1_ref[p, 0] + b1_ref[p], 0.0) for p in range(3)]
    h1 = [_r(h) for h in h1]
    h2 = []
    for g in range(5):
        t = (h1[0] * w2_ref[g, 0] + h1[1] * w2_ref[g, 1]
             + h1[2] * w2_ref[g, 2] + b2_ref[g])
        h2.append(jnp.maximum(t, 0.0))
    h2 = [_r(h) for h in h2]
    gv = (h2[0] * w3_ref[0, 0] + h2[1] * w3_ref[0, 1] + h2[2] * w3_ref[0, 2]
          + h2[3] * w3_ref[0, 3] + h2[4] * w3_ref[0, 4] + b3_ref[0])
    mu = jnp.mean(gv, axis=1, keepdims=True)
    dev = gv - mu
    var = jnp.mean(dev * dev, axis=1, keepdims=True)
    gn_ref[...] = _r(dev / jnp.sqrt(var + 1e-5))


def _gn_call(ev2, w1, b1, w2, b2, w3, b3):
    smem = pl.BlockSpec(memory_space=pltpu.SMEM)
    return pl.pallas_call(
        _gn_body,
        in_specs=[pl.BlockSpec((_L, _N), lambda: (0, 0)),
                  smem, smem, smem, smem, smem, smem],
        out_specs=pl.BlockSpec((_L, _N), lambda: (0, 0)),
        out_shape=jax.ShapeDtypeStruct((_L, _N), jnp.float32),
    )(ev2, w1, b1, w2, b2, w3, b3)


def _mv_body(gn_ref, e_ref, f_ref):
    l = pl.program_id(0)
    gnb = gn_ref[pl.ds(l, 1), :].astype(jnp.bfloat16)
    f_ref[pl.ds(l, 1), :] = jax.lax.dot_general(
        gnb, e_ref[0].astype(jnp.bfloat16), (((1,), (1,)), ((), ())),
        preferred_element_type=jnp.float32)


def _mv_call(gn, evecs):
    return pl.pallas_call(
        _mv_body,
        grid=(_L,),
        in_specs=[
            pl.BlockSpec((_L, _N), lambda l: (0, 0)),
            pl.BlockSpec((1, _M0, _N), lambda l: (l, 0, 0)),
        ],
        out_specs=pl.BlockSpec((_L, _M0), lambda l: (0, 0)),
        out_shape=jax.ShapeDtypeStruct((_L, _M0), jnp.float32),
    )(gn, evecs)


def _scmv_body(e_hbm, gn_hbm, out_hbm, gn_v, eb0, eb1, o_v, sem0, sem1):
    c_ = jax.lax.axis_index("c")
    s_ = jax.lax.axis_index("s")
    wid = s_ * _NC + c_
    pltpu.sync_copy(gn_hbm.at[wid], gn_v)
    lanes = jax.lax.iota(jnp.int32, _LANES)
    nch = _RH // _W
    bufs = (eb0, eb1)
    sems = (sem0, sem1)
    cps = [None, None]
    cps[0] = pltpu.async_copy(
        e_hbm.at[wid, pl.ds(_M0, _W), :], eb0, sem0)
    for c in range(nch):
        if c + 1 < nch:
            cps[(c + 1) % 2] = pltpu.async_copy(
                e_hbm.at[wid, pl.ds(_M0 + (c + 1) * _W, _W), :],
                bufs[(c + 1) % 2], sems[(c + 1) % 2])
        cps[c % 2].wait()
        ebuf = bufs[c % 2]

        res = jnp.zeros((_LANES,), jnp.float32)
        for rg in range(_W // 4):
            def inner(j, accs, rg=rg):
                a0, a1, a2, a3 = accs
                g = gn_v[pl.ds(j * _LANES, _LANES)]
                sl = pl.ds(j * _LANES, _LANES)
                a0 = a0 + _r(ebuf[rg * 4 + 0, sl]) * g
                a1 = a1 + _r(ebuf[rg * 4 + 1, sl]) * g
                a2 = a2 + _r(ebuf[rg * 4 + 2, sl]) * g
                a3 = a3 + _r(ebuf[rg * 4 + 3, sl]) * g
                return a0, a1, a2, a3

            z = jnp.zeros((_LANES,), jnp.float32)
            a0, a1, a2, a3 = jax.lax.fori_loop(
                0, _N // _LANES, inner, (z, z, z, z))
            for t, a in enumerate((a0, a1, a2, a3)):
                res = jnp.where(lanes == rg * 4 + t, jnp.sum(a), res)
        o_v[pl.ds(c * _W, _W)] = res
    pltpu.sync_copy(o_v, out_hbm.at[wid])


def _scmv_call(evecs, gn):
    mesh = plsc.VectorSubcoreMesh(core_axis_name="c", subcore_axis_name="s")
    run = functools.partial(
        pl.kernel,
        out_type=jax.ShapeDtypeStruct((_L, _RH), jnp.float32),
        mesh=mesh,
        compiler_params=pltpu.CompilerParams(needs_layout_passes=False),
        scratch_types=[
            pltpu.VMEM((_N,), jnp.float32),
            pltpu.VMEM((_W, _N), jnp.float32),
            pltpu.VMEM((_W, _N), jnp.float32),
            pltpu.VMEM((_RH,), jnp.float32),
            pltpu.SemaphoreType.DMA,
            pltpu.SemaphoreType.DMA,
        ],
    )(_scmv_body)
    return run(evecs, gn)


def _sc_body(flo_hbm, fhi_hbm, bi_hbm, di_hbm, wpb_hbm, out_hbm,
             f_v, bi_v, di_v, b_v, d_v, wp_v, o_v):
    c = jax.lax.axis_index("c")
    s = jax.lax.axis_index("s")
    wid = s * _NC + c
    pltpu.sync_copy(flo_hbm.at[wid], f_v.at[pl.ds(0, _M0)])
    pltpu.sync_copy(fhi_hbm.at[wid], f_v.at[pl.ds(_M0, _RH)])
    pltpu.sync_copy(bi_hbm.at[wid], bi_v)
    pltpu.sync_copy(di_hbm.at[wid], di_v)
    pltpu.sync_copy(wpb_hbm, wp_v)

    wpvec = wp_v[...]
    lanes = jax.lax.iota(jnp.int32, _LANES)
    neg_inf = jnp.full((_LANES,), -jnp.inf, jnp.float32)
    nchunks = _M // _LANES
    acc = jnp.zeros((_LANES,), jnp.float32)

    for plane in range(_PLANES):
        p_regs = []
        for j in range(nchunks):
            off = plane * _M + j * _LANES
            bi = bi_v[pl.ds(off, _LANES)]
            di = di_v[pl.ds(off, _LANES)]
            bvals = plsc.load_gather(f_v, [bi])
            dvals = plsc.load_gather(f_v, [di])
            b_v[pl.ds(j * _LANES, _LANES)] = bvals
            d_v[pl.ds(j * _LANES, _LANES)] = dvals
            p_regs.append(jnp.abs(dvals - bvals))

        def step(k, carry):
            ps = list(carry[:nchunks])
            prev_b, prev_d, first_b, first_d, cross = carry[nchunks:]
            m = ps[0]
            for j in range(1, nchunks):
                m = jnp.maximum(m, ps[j])
            mb = jnp.broadcast_to(jnp.max(m), (_LANES,))
            selv = jnp.full((_LANES,), 16384, jnp.int32)
            for j in range(nchunks):
                cand = jnp.where(ps[j] == mb, lanes + (j * _LANES), selv)
                selv = jnp.minimum(selv, cand)
            selb = jnp.broadcast_to(jnp.min(selv), (_LANES,))
            bs = plsc.load_gather(b_v, [selb])
            ds_ = plsc.load_gather(d_v, [selb])
            for j in range(nchunks):
                hit = (lanes + (j * _LANES)) == selb
                ps[j] = jnp.where(hit, neg_inf, ps[j])
            cross = cross + bs * prev_d - ds_ * prev_b
            is0 = jnp.broadcast_to(k == 0, (_LANES,))
            first_b = jnp.where(is0, bs, first_b)
            first_d = jnp.where(is0, ds_, first_d)
            return tuple(ps) + (bs, ds_, first_b, first_d, cross)

        z = jnp.zeros((_LANES,), jnp.float32)
        fin = jax.lax.fori_loop(0, _K, step, tuple(p_regs) + (z, z, z, z, z))
        last_b, last_d, first_b, first_d, cross = fin[nchunks:]
        incx = first_b - last_b
        incy = first_d - last_d
        area = 0.5 * (cross - (last_b * first_d - last_d * first_b))
        acc = (acc + incx * wpvec[3 * plane] + incy * wpvec[3 * plane + 1]
               + area * wpvec[3 * plane + 2])

    o_v[...] = acc + wpvec[3 * _PLANES]
    pltpu.sync_copy(o_v, out_hbm.at[wid])


def _sc_call(f_lo, f_hi, bidx2, didx2, wpb):
    mesh = plsc.VectorSubcoreMesh(core_axis_name="c", subcore_axis_name="s")
    run = functools.partial(
        pl.kernel,
        out_type=jax.ShapeDtypeStruct((_L, _LANES), jnp.float32),
        mesh=mesh,
        compiler_params=pltpu.CompilerParams(needs_layout_passes=False),
        scratch_types=[
            pltpu.VMEM((_N,), jnp.float32),
            pltpu.VMEM((_PLANES * _M,), jnp.int32),
            pltpu.VMEM((_PLANES * _M,), jnp.int32),
            pltpu.VMEM((_M,), jnp.float32),
            pltpu.VMEM((_M,), jnp.float32),
            pltpu.VMEM((_LANES,), jnp.float32),
            pltpu.VMEM((_LANES,), jnp.float32),
        ],
    )(_sc_body)
    return run(f_lo, f_hi, bidx2, didx2, wpb)


def kernel(eigenvalues, eigenvectors_sq, W1, b1, W2, b2, W3, b3, Wp, bp,
           birth_idx, death_idx):
    ev2 = eigenvalues.reshape(_L, _N)
    gn = _gn_call(ev2, W1, b1, _r(W2), b2, _r(W3), b3)
    f_lo = _mv_call(gn, eigenvectors_sq)
    f_hi = _scmv_call(eigenvectors_sq, gn)
    bidx2 = birth_idx.reshape(_L, _PLANES * _M)
    didx2 = death_idx.reshape(_L, _PLANES * _M)
    wpb = jnp.concatenate([Wp.reshape(-1), bp.reshape(-1),
                           jnp.zeros(_LANES - 3 * _PLANES - 1, jnp.float32)])
    o = _sc_call(f_lo, f_hi, bidx2, didx2, wpb)
    return o[:, :1]
